# baseline (speedup 1.0000x reference)
import functools

import jax
import jax.numpy as jnp
from jax import lax
from jax.experimental import pallas as pl
from jax.experimental.pallas import tpu as pltpu
from jax.experimental.pallas import tpu_sc as plsc

_MAX_LEN = 8192
_DIM = 1024
_OUT_LEN = 4096
_NC = 2
_NS = 16
_L = 16
_NW = _NC * _NS
_ROWS_PER_W = _OUT_LEN // _NW
_CHUNK = 16
_NCHUNK = _ROWS_PER_W // _CHUNK
_NBUF = 6
_AHEAD = 3

_mesh = plsc.VectorSubcoreMesh(core_axis_name="c", subcore_axis_name="s")


@functools.partial(
    pl.kernel,
    mesh=_mesh,
    out_type=jax.ShapeDtypeStruct((_OUT_LEN, _DIM), jnp.float32),
    scratch_types=(
        [pltpu.VMEM((_L,), jnp.int32)]
        + [pltpu.VMEM((_CHUNK, _DIM), jnp.float32) for _ in range(_NBUF)]
        + [pltpu.SemaphoreType.DMA for _ in range(2 * _NBUF + 1)]
    ),
)
def _sc_slice_copy(len_hbm, table_hbm, out_hbm, len_v, *bufs_and_sems):
    bufs = bufs_and_sems[:_NBUF]
    sins = bufs_and_sems[_NBUF:2 * _NBUF]
    souts = bufs_and_sems[2 * _NBUF:3 * _NBUF]
    lsem = bufs_and_sems[3 * _NBUF]

    wid = lax.axis_index("s") * _NC + lax.axis_index("c")
    base = wid * _ROWS_PER_W

    ins = []
    outs = []
    for c in range(_NCHUNK):
        cb = base + c * _CHUNK
        b = c % _NBUF
        ins.append(pltpu.make_async_copy(
            table_hbm.at[pl.ds(cb, _CHUNK)], bufs[b], sins[b]))
        outs.append(pltpu.make_async_copy(
            bufs[b], out_hbm.at[pl.ds(cb, _CHUNK)], souts[b]))

    for c in range(_AHEAD):
        ins[c].start()
    lcopy = pltpu.make_async_copy(len_hbm, len_v, lsem)
    lcopy.start()
    lcopy.wait()
    length = len_v[...][0]

    zero = jnp.zeros((_L,), jnp.float32)

    for c in range(_NCHUNK):
        ins[c].wait()

        cb = base + c * _CHUNK
        nvalid = jnp.clip(length - cb, 0, _CHUNK)
        buf = bufs[c % _NBUF]

        @pl.when(nvalid < _CHUNK)
        def _():
            def zero_row(r, carry):
                for j in range(_DIM // _L):
                    buf[r, pl.ds(j * _L, _L)] = zero
                return carry
            lax.fori_loop(nvalid, _CHUNK, zero_row, 0)

        outs[c].start()
        nxt = c + _AHEAD
        if nxt < _NCHUNK:
            prev = nxt - _NBUF
            if prev >= 0:
                outs[prev].wait()
            ins[nxt].start()
    for c in range(max(_NCHUNK - _NBUF, 0), _NCHUNK):
        outs[c].wait()


def kernel(length, pos_embedding):
    len_arr = jnp.broadcast_to(jnp.asarray(length, jnp.int32), (_L,))
    return _sc_slice_copy(len_arr, pos_embedding)


# speedup vs baseline: 1.0062x; 1.0062x over previous
import functools

import jax
import jax.numpy as jnp
from jax import lax
from jax.experimental import pallas as pl
from jax.experimental.pallas import tpu as pltpu
from jax.experimental.pallas import tpu_sc as plsc

_MAX_LEN = 8192
_DIM = 1024
_OUT_LEN = 4096
_NC = 2
_NS = 16
_L = 16
_NW = _NC * _NS
_ROWS_PER_W = _OUT_LEN // _NW
_CHUNK = 32
_NCHUNK = _ROWS_PER_W // _CHUNK
_NBUF = 3
_AHEAD = 2

_mesh = plsc.VectorSubcoreMesh(core_axis_name="c", subcore_axis_name="s")


@functools.partial(
    pl.kernel,
    mesh=_mesh,
    out_type=jax.ShapeDtypeStruct((_OUT_LEN, _DIM), jnp.float32),
    scratch_types=(
        [pltpu.VMEM((_L,), jnp.int32)]
        + [pltpu.VMEM((_CHUNK, _DIM), jnp.float32) for _ in range(_NBUF)]
        + [pltpu.SemaphoreType.DMA for _ in range(2 * _NBUF + 1)]
    ),
)
def _sc_slice_copy(len_hbm, table_hbm, out_hbm, len_v, *bufs_and_sems):
    bufs = bufs_and_sems[:_NBUF]
    sins = bufs_and_sems[_NBUF:2 * _NBUF]
    souts = bufs_and_sems[2 * _NBUF:3 * _NBUF]
    lsem = bufs_and_sems[3 * _NBUF]

    wid = lax.axis_index("s") * _NC + lax.axis_index("c")
    base = wid * _ROWS_PER_W

    ins = []
    outs = []
    for c in range(_NCHUNK):
        cb = base + c * _CHUNK
        b = c % _NBUF
        ins.append(pltpu.make_async_copy(
            table_hbm.at[pl.ds(cb, _CHUNK)], bufs[b], sins[b]))
        outs.append(pltpu.make_async_copy(
            bufs[b], out_hbm.at[pl.ds(cb, _CHUNK)], souts[b]))

    for c in range(_AHEAD):
        ins[c].start()
    lcopy = pltpu.make_async_copy(len_hbm, len_v, lsem)
    lcopy.start()
    lcopy.wait()
    length = len_v[...][0]

    zero = jnp.zeros((_L,), jnp.float32)

    for c in range(_NCHUNK):
        ins[c].wait()

        cb = base + c * _CHUNK
        nvalid = jnp.clip(length - cb, 0, _CHUNK)
        buf = bufs[c % _NBUF]

        @pl.when(nvalid < _CHUNK)
        def _():
            def zero_row(r, carry):
                for j in range(_DIM // _L):
                    buf[r, pl.ds(j * _L, _L)] = zero
                return carry
            lax.fori_loop(nvalid, _CHUNK, zero_row, 0)

        outs[c].start()
        nxt = c + _AHEAD
        if nxt < _NCHUNK:
            prev = nxt - _NBUF
            if prev >= 0:
                outs[prev].wait()
            ins[nxt].start()
    for c in range(max(_NCHUNK - _NBUF, 0), _NCHUNK):
        outs[c].wait()


def kernel(length, pos_embedding):
    len_arr = jnp.broadcast_to(jnp.asarray(length, jnp.int32), (_L,))
    return _sc_slice_copy(len_arr, pos_embedding)
